# Initial kernel scaffold; baseline (speedup 1.0000x reference)
import jax
import jax.numpy as jnp
from jax.experimental import pallas as pl


def kernel(input, weight):
    raise NotImplementedError("write your pallas kernel here")


# speedup vs baseline: 1.5478x; 1.5478x over previous
import functools

import jax
import jax.numpy as jnp
from jax import lax
from jax.experimental import pallas as pl
from jax.experimental.pallas import tpu as pltpu
from jax.experimental.pallas import tpu_sc as plsc

_NC = 2
_NS = 16
_NW = _NC * _NS

_D = 32
_IPR = 128
_ROWS_PER_CHUNK = 8
_CHUNK = _ROWS_PER_CHUNK * _IPR


def _gather_body(table_hbm, idx_hbm, out_hbm, idx_v, rows_v, sem, *,
                 rows_per_w):
  wid = lax.axis_index("s") * _NC + lax.axis_index("c")
  row0 = wid * rows_per_w
  n_chunks = rows_per_w // _ROWS_PER_CHUNK

  def chunk_body(g, carry):
    r = row0 + g * _ROWS_PER_CHUNK
    pltpu.sync_copy(idx_hbm.at[pl.ds(r, _ROWS_PER_CHUNK)], idx_v)
    copies = [
        pltpu.async_copy(
            table_hbm.at[idx_v.at[j]],
            rows_v.at[pl.ds(j * _IPR, _IPR)],
            sem,
        )
        for j in range(_ROWS_PER_CHUNK)
    ]
    for cp in copies:
      cp.wait()
    pltpu.sync_copy(rows_v, out_hbm.at[pl.ds(r * _IPR, _CHUNK)])
    return carry

  lax.fori_loop(0, n_chunks, chunk_body, 0)


@functools.partial(jax.jit, static_argnames=())
def _sc_gather(table, idx2d):
  n_rows = idx2d.shape[0]
  rows_per_w = n_rows // _NW
  b = n_rows * _IPR
  mesh = plsc.VectorSubcoreMesh(core_axis_name="c", subcore_axis_name="s")
  body = functools.partial(_gather_body, rows_per_w=rows_per_w)
  return pl.kernel(
      body,
      out_type=jax.ShapeDtypeStruct((b, _D), jnp.float32),
      mesh=mesh,
      scratch_types=[
          pltpu.VMEM((_ROWS_PER_CHUNK, _IPR), jnp.int32),
          pltpu.VMEM((_CHUNK, _D), jnp.float32),
          pltpu.SemaphoreType.DMA,
      ],
      compiler_params=pltpu.CompilerParams(use_tc_tiling_on_sc=False),
  )(table, idx2d)


def kernel(input, weight):
  b = input.size
  idx2d = input.reshape(b // _IPR, _IPR)
  out = _sc_gather(weight, idx2d)
  return out.reshape(input.shape + (weight.shape[1],))


# speedup vs baseline: 1.5499x; 1.0014x over previous
import functools

import jax
import jax.numpy as jnp
from jax import lax
from jax.experimental import pallas as pl
from jax.experimental.pallas import tpu as pltpu
from jax.experimental.pallas import tpu_sc as plsc

_NC = 2
_NS = 16
_NW = _NC * _NS

_D = 32
_IPR = 128
_CR = 4
_CHUNK = _CR * _IPR
_NBUF = 2


def _gather_body(table_hbm, idx_hbm, out_hbm, idx_v, rows_v, idx_sem,
                 gat_sem, out_sem, *, rows_per_w):
  wid = lax.axis_index("s") * _NC + lax.axis_index("c")
  row0 = wid * rows_per_w
  n_chunks = rows_per_w // _CR

  def idx_copy(g, q):
    return pltpu.make_async_copy(
        idx_hbm.at[pl.ds(row0 + g * _CR, _CR)], idx_v.at[q], idx_sem.at[q])

  def out_copy(g, q):
    return pltpu.make_async_copy(
        rows_v.at[q], out_hbm.at[pl.ds((row0 + g * _CR) * _IPR, _CHUNK)],
        out_sem.at[q])

  def gather_copies(q):
    return [
        pltpu.make_async_copy(
            table_hbm.at[idx_v.at[q].at[j]],
            rows_v.at[q].at[pl.ds(j * _IPR, _IPR)],
            gat_sem,
        )
        for j in range(_CR)
    ]

  for q in range(_NBUF):
    idx_copy(q, q).start()
    out_copy(q, q).start()

  def chunk_pair(t, carry):
    for q in range(_NBUF):
      g = t * _NBUF + q
      out_copy(g, q).wait()
      idx_copy(g, q).wait()
      for cp in gather_copies(q):
        cp.start()
      for cp in gather_copies(q):
        cp.wait()
      g_next = jnp.minimum(g + _NBUF, n_chunks - 1)
      idx_copy(g_next, q).start()
      out_copy(g, q).start()
    return carry

  lax.fori_loop(0, n_chunks // _NBUF, chunk_pair, 0)

  for q in range(_NBUF):
    out_copy(n_chunks - _NBUF + q, q).wait()
    idx_copy(n_chunks - 1, q).wait()


@jax.jit
def _sc_gather(table, idx2d):
  n_rows = idx2d.shape[0]
  rows_per_w = n_rows // _NW
  b = n_rows * _IPR
  mesh = plsc.VectorSubcoreMesh(core_axis_name="c", subcore_axis_name="s")
  body = functools.partial(_gather_body, rows_per_w=rows_per_w)
  return pl.kernel(
      body,
      out_type=jax.ShapeDtypeStruct((b, _D), jnp.float32),
      mesh=mesh,
      scratch_types=[
          pltpu.VMEM((_NBUF, _CR, _IPR), jnp.int32),
          pltpu.VMEM((_NBUF, _CHUNK, _D), jnp.float32),
          pltpu.SemaphoreType.DMA((_NBUF,)),
          pltpu.SemaphoreType.DMA,
          pltpu.SemaphoreType.DMA((_NBUF,)),
      ],
      compiler_params=pltpu.CompilerParams(use_tc_tiling_on_sc=False),
  )(table, idx2d)


def kernel(input, weight):
  b = input.size
  idx2d = input.reshape(b // _IPR, _IPR)
  out = _sc_gather(weight, idx2d)
  return out.reshape(input.shape + (weight.shape[1],))
